# baseline (speedup 1.0000x reference)
import functools

import jax
import jax.numpy as jnp
from jax import lax
from jax.experimental import pallas as pl
from jax.experimental.pallas import tpu as pltpu
from jax.experimental.pallas import tpu_sc as plsc

VOCAB_SIZE = 1000


def _build_sc_lookup(B, V, D, Dp, NC, NS, L):
    NW = NC * NS
    b_per_w = B // NW
    C = 1280
    assert b_per_w % C == 0 and C % 2 == 0
    n_chunks = b_per_w // C
    CW = C * D

    mesh = plsc.VectorSubcoreMesh(core_axis_name="c", subcore_axis_name="s")

    @functools.partial(
        pl.kernel,
        mesh=mesh,
        compiler_params=pltpu.CompilerParams(use_tc_tiling_on_sc=False),
        out_type=jax.ShapeDtypeStruct((B * D,), jnp.float32),
        scratch_types=[
            pltpu.VMEM((C,), jnp.int32),
            pltpu.VMEM((C, Dp), jnp.float32),
            pltpu.VMEM((CW + L,), jnp.float32),
            pltpu.SemaphoreType.DMA,
        ],
    )
    def sc_lookup(idx_hbm, table_hbm, out_hbm, idx_v, rows_v, comp_v, sem):
        wid = lax.axis_index("s") * NC + lax.axis_index("c")
        base = wid * b_per_w

        def chunk_body(ci, _):
            off = base + ci * C
            pltpu.sync_copy(idx_hbm.at[pl.ds(off, C)], idx_v)

            def remap(j, _):
                v = idx_v[pl.ds(j * L, L)]
                ok = (v >= 0) & (v < V)
                idx_v[pl.ds(j * L, L)] = jnp.where(ok, v + 1, jnp.zeros_like(v))
                return 0

            lax.fori_loop(0, C // L, remap, 0, unroll=4)

            pltpu.async_copy(table_hbm.at[idx_v], rows_v, sem).wait()

            def pack(i, _):
                dst = i * D
                comp_v[pl.ds(dst, L)] = rows_v[i, pl.ds(0, L)]
                comp_v[pl.ds(dst + L, L)] = rows_v[i, pl.ds(L, L)]
                comp_v[pl.ds(dst + Dp - L, L)] = rows_v[i, pl.ds(Dp - L, L)]
                return 0

            lax.fori_loop(0, C, pack, 0)

            pltpu.sync_copy(comp_v.at[pl.ds(0, CW)],
                            out_hbm.at[pl.ds(off * D, CW)])
            return 0

        lax.fori_loop(0, n_chunks, chunk_body, 0)

    return sc_lookup


def kernel(indices, table):
    B = indices.shape[0] * indices.shape[1]
    V = VOCAB_SIZE
    D = table.shape[1]
    Dp = (D + 7) // 8 * 8
    info = plsc.get_sparse_core_info()
    NC, NS, L = info.num_cores, info.num_subcores, info.num_lanes
    sc_lookup = _build_sc_lookup(B, V, D, Dp, NC, NS, L)
    flat_idx = indices.reshape(B)
    table_p = jnp.pad(table, ((0, 0), (0, Dp - D)))
    out = sc_lookup(flat_idx, table_p)
    return out.reshape(indices.shape[0], indices.shape[1], D)


# speedup vs baseline: 1.1648x; 1.1648x over previous
import functools

import jax
import jax.numpy as jnp
from jax import lax
from jax.experimental import pallas as pl
from jax.experimental.pallas import tpu as pltpu
from jax.experimental.pallas import tpu_sc as plsc

VOCAB_SIZE = 1000


def _build_sc_lookup(R, S, V, D, Dp, NC, NS, L):
    NW = NC * NS
    B = R * S
    r_per_w = R // NW
    CR = 8
    C = CR * S
    n_chunks = r_per_w // CR

    mesh = plsc.VectorSubcoreMesh(core_axis_name="c", subcore_axis_name="s")

    @functools.partial(
        pl.kernel,
        mesh=mesh,
        compiler_params=pltpu.CompilerParams(use_tc_tiling_on_sc=False),
        out_type=jax.ShapeDtypeStruct((R, S, D), jnp.float32),
        scratch_types=[
            pltpu.VMEM((C,), jnp.int32),
            pltpu.VMEM((C, Dp), jnp.float32),
            pltpu.VMEM((CR, S, D), jnp.float32),
            pltpu.SemaphoreType.DMA,
        ],
    )
    def sc_lookup(idx_hbm, table_hbm, out_hbm, idx_v, rows_v, comp_v, sem):
        wid = lax.axis_index("s") * NC + lax.axis_index("c")
        row_base = wid * r_per_w

        def chunk_body(ci, _):
            row0 = row_base + ci * CR
            off = row0 * S
            pltpu.sync_copy(idx_hbm.at[pl.ds(off, C)], idx_v)

            def remap(j, _):
                v = idx_v[pl.ds(j * L, L)]
                ok = (v >= 0) & (v < V)
                idx_v[pl.ds(j * L, L)] = jnp.where(ok, v + 1, jnp.zeros_like(v))
                return 0

            lax.fori_loop(0, C // L, remap, 0, unroll=4)

            pltpu.async_copy(table_hbm.at[idx_v], rows_v, sem).wait()

            def pack_outer(a, _):
                def pack_inner(b, _):
                    i = a * S + b
                    comp_v[a, b, pl.ds(0, L)] = rows_v[i, pl.ds(0, L)]
                    comp_v[a, b, pl.ds(L, L)] = rows_v[i, pl.ds(L, L)]
                    comp_v[a, b, pl.ds(D - L, L)] = rows_v[i, pl.ds(D - L, L)]
                    return 0

                lax.fori_loop(0, S, pack_inner, 0, unroll=4)
                return 0

            lax.fori_loop(0, CR, pack_outer, 0)

            pltpu.sync_copy(comp_v, out_hbm.at[pl.ds(row0, CR)])
            return 0

        lax.fori_loop(0, n_chunks, chunk_body, 0)

    return sc_lookup


def kernel(indices, table):
    R, S = indices.shape
    V = VOCAB_SIZE
    D = table.shape[1]
    Dp = (D + 7) // 8 * 8
    info = plsc.get_sparse_core_info()
    NC, NS, L = info.num_cores, info.num_subcores, info.num_lanes
    sc_lookup = _build_sc_lookup(R, S, V, D, Dp, NC, NS, L)
    flat_idx = indices.reshape(R * S)
    table_p = jnp.pad(table, ((0, 0), (0, Dp - D)))
    return sc_lookup(flat_idx, table_p)


# speedup vs baseline: 2.2797x; 1.9572x over previous
import functools

import jax
import jax.numpy as jnp
from jax import lax
from jax.experimental import pallas as pl
from jax.experimental.pallas import tpu as pltpu
from jax.experimental.pallas import tpu_sc as plsc

VOCAB_SIZE = 1000


def _build_sc_lookup(R, S, V, D, Dp, NC, NS, L):
    NW = NC * NS
    ST = S // 8
    CT = 1024

    mesh = plsc.VectorSubcoreMesh(core_axis_name="c", subcore_axis_name="s")

    @functools.partial(
        pl.kernel,
        mesh=mesh,
        compiler_params=pltpu.CompilerParams(
            use_tc_tiling_on_sc=False, needs_layout_passes=False),
        out_type=jax.ShapeDtypeStruct((D * ST * NW * CT,), jnp.float32),
        scratch_types=[
            pltpu.VMEM((CT,), jnp.int32),
            pltpu.VMEM((CT, Dp), jnp.float32),
            pltpu.VMEM((D, CT), jnp.float32),
            pltpu.SemaphoreType.DMA,
        ],
    )
    def sc_lookup(idx_hbm, table_hbm, out_hbm, idx_v, rows_v, trans_v, sem):
        wid = lax.axis_index("s") * NC + lax.axis_index("c")

        def chunk_body(st, _):
            off = (wid * ST + st) * CT
            pltpu.sync_copy(idx_hbm.at[pl.ds(off, CT)], idx_v)

            def remap(j, _):
                v = idx_v[pl.ds(j * L, L)]
                ok = (v >= 0) & (v < V)
                idx_v[pl.ds(j * L, L)] = jnp.where(ok, v + 1, jnp.zeros_like(v))
                return 0

            lax.fori_loop(0, CT // L, remap, 0, unroll=4)

            pltpu.async_copy(table_hbm.at[idx_v], rows_v, sem).wait()

            lanes = lax.iota(jnp.int32, L)

            def trans_d(d, _):
                dcol = jnp.full((L,), d, jnp.int32)

                def trans_k(k, _):
                    rowv = lanes + k * L
                    trans_v[d, pl.ds(k * L, L)] = plsc.load_gather(
                        rows_v, [rowv, dcol])
                    return 0

                lax.fori_loop(0, CT // L, trans_k, 0, unroll=4)
                return 0

            lax.fori_loop(0, D, trans_d, 0)

            def dma_d(d, _):
                dst = ((d * ST + st) * NW + wid) * CT
                pltpu.sync_copy(trans_v.at[d], out_hbm.at[pl.ds(dst, CT)])
                return 0

            lax.fori_loop(0, D, dma_d, 0)
            return 0

        lax.fori_loop(0, ST, chunk_body, 0)

    return sc_lookup


def kernel(indices, table):
    R, S = indices.shape
    V = VOCAB_SIZE
    D = table.shape[1]
    Dp = (D + 7) // 8 * 8
    info = plsc.get_sparse_core_info()
    NC, NS, L = info.num_cores, info.num_subcores, info.num_lanes
    NW = NC * NS
    ST = S // 8
    sc_lookup = _build_sc_lookup(R, S, V, D, Dp, NC, NS, L)
    idx_r = (indices.reshape(NW, 128, ST, 8)
             .transpose(0, 2, 3, 1)
             .reshape(R * S))
    table_p = jnp.pad(table, ((0, 0), (0, Dp - D)))
    out5 = sc_lookup(idx_r, table_p).reshape(D, ST, NW, 8, 128)
    return out5.transpose(2, 4, 1, 3, 0).reshape(R, S, D)


# speedup vs baseline: 4.0752x; 1.7876x over previous
import functools

import jax
import jax.numpy as jnp
from jax import lax
from jax.experimental import pallas as pl
from jax.experimental.pallas import tpu as pltpu
from jax.experimental.pallas import tpu_sc as plsc

VOCAB_SIZE = 1000


def _build_sc_lookup(R, S, V, D, NC, NS, L):
    NW = NC * NS
    ST = S // 8
    CT = 1024
    TBW = ((V + 1) * D + 7) // 8 * 8

    mesh = plsc.VectorSubcoreMesh(core_axis_name="c", subcore_axis_name="s")

    @functools.partial(
        pl.kernel,
        mesh=mesh,
        compiler_params=pltpu.CompilerParams(
            use_tc_tiling_on_sc=False, needs_layout_passes=False),
        out_type=jax.ShapeDtypeStruct((D * ST * NW * CT,), jnp.float32),
        scratch_types=[
            pltpu.VMEM((TBW,), jnp.float32),
            pltpu.VMEM((CT,), jnp.int32),
            pltpu.VMEM((D, CT), jnp.float32),
            pltpu.VMEM((D, CT), jnp.float32),
            pltpu.SemaphoreType.DMA,
            pltpu.SemaphoreType.DMA,
        ],
    )
    def sc_lookup(idx_hbm, table_hbm, out_hbm, table_v, idx_v,
                  trans_a, trans_b, sem_a, sem_b):
        wid = lax.axis_index("s") * NC + lax.axis_index("c")
        pltpu.sync_copy(table_hbm, table_v)

        def compute_chunk(st, trans_v):
            off = (wid * ST + st) * CT
            pltpu.sync_copy(idx_hbm.at[pl.ds(off, CT)], idx_v)

            def kbody(k, _):
                v = idx_v[pl.ds(k * L, L)]
                ok = (v >= 0) & (v < V)
                base = jnp.where(ok, v + 1, jnp.zeros_like(v)) * D
                for d in range(D):
                    trans_v[d, pl.ds(k * L, L)] = plsc.load_gather(
                        table_v, [base + d])
                return 0

            lax.fori_loop(0, CT // L, kbody, 0)

        def fire_chunk(st, trans_v, sem):
            def dma_d(d, _):
                dst = ((d * ST + st) * NW + wid) * CT
                pltpu.async_copy(trans_v.at[d], out_hbm.at[pl.ds(dst, CT)], sem)
                return 0

            lax.fori_loop(0, D, dma_d, 0)

        def drain(trans_v, sem):
            pltpu.make_async_copy(
                out_hbm.at[pl.ds(0, D * CT)], trans_v, sem).wait()

        def pair(p, _):
            @pl.when(p > 0)
            def _():
                drain(trans_a, sem_a)

            compute_chunk(2 * p, trans_a)
            fire_chunk(2 * p, trans_a, sem_a)

            @pl.when(p > 0)
            def _():
                drain(trans_b, sem_b)

            compute_chunk(2 * p + 1, trans_b)
            fire_chunk(2 * p + 1, trans_b, sem_b)
            return 0

        lax.fori_loop(0, ST // 2, pair, 0)
        drain(trans_a, sem_a)
        compute_chunk(ST - 1, trans_a)
        fire_chunk(ST - 1, trans_a, sem_a)
        drain(trans_a, sem_a)
        drain(trans_b, sem_b)

    return sc_lookup


def kernel(indices, table):
    R, S = indices.shape
    V = VOCAB_SIZE
    D = table.shape[1]
    info = plsc.get_sparse_core_info()
    NC, NS, L = info.num_cores, info.num_subcores, info.num_lanes
    NW = NC * NS
    ST = S // 8
    sc_lookup = _build_sc_lookup(R, S, V, D, NC, NS, L)
    idx_r = (indices.reshape(NW, 128, ST, 8)
             .transpose(0, 2, 3, 1)
             .reshape(R * S))
    TBW = ((V + 1) * D + 7) // 8 * 8
    table_f = jnp.pad(table.reshape(-1), (0, TBW - (V + 1) * D))
    out5 = sc_lookup(idx_r, table_f).reshape(D, ST, NW, 8, 128)
    return out5.transpose(2, 4, 1, 3, 0).reshape(R, S, D)


# speedup vs baseline: 9.5741x; 2.3494x over previous
import functools

import jax
import jax.numpy as jnp
from jax import lax
from jax.experimental import pallas as pl
from jax.experimental.pallas import tpu as pltpu
from jax.experimental.pallas import tpu_sc as plsc

VOCAB_SIZE = 1000


def _build_sc_lookup(R, S, V, D, NC, NS, L):
    NW = NC * NS
    ST = S // 8
    CT = 1024
    TBW = ((V + 1) * D + 7) // 8 * 8

    mesh = plsc.VectorSubcoreMesh(core_axis_name="c", subcore_axis_name="s")

    @functools.partial(
        pl.kernel,
        mesh=mesh,
        compiler_params=pltpu.CompilerParams(
            use_tc_tiling_on_sc=False, needs_layout_passes=False),
        out_type=jax.ShapeDtypeStruct((D * ST * NW * CT,), jnp.float32),
        scratch_types=[
            pltpu.VMEM((TBW,), jnp.float32),
            pltpu.VMEM((CT,), jnp.int32),
            pltpu.VMEM((D, CT), jnp.float32),
            pltpu.VMEM((D, CT), jnp.float32),
            pltpu.SemaphoreType.DMA,
            pltpu.SemaphoreType.DMA,
        ],
    )
    def sc_lookup(idx_hbm, table_hbm, out_hbm, table_v, idx_v,
                  trans_a, trans_b, sem_a, sem_b):
        wid = lax.axis_index("s") * NC + lax.axis_index("c")
        pltpu.sync_copy(table_hbm, table_v)

        def compute_chunk(st, trans_v):
            off = (wid * ST + st) * CT
            pltpu.sync_copy(idx_hbm.at[pl.ds(off, CT)], idx_v)

            def kbody(k, _):
                v = idx_v[pl.ds(k * L, L)]
                ok = (v >= 0) & (v < V)
                base = jnp.where(ok, v + 1, jnp.zeros_like(v)) * D
                vals = [plsc.load_gather(table_v, [base + d])
                        for d in range(D)]
                for d in range(D):
                    trans_v[d, pl.ds(k * L, L)] = vals[d]
                return 0

            lax.fori_loop(0, CT // L, kbody, 0)

        def fire_chunk(st, trans_v, sem):
            def dma_d(d, _):
                dst = ((d * ST + st) * NW + wid) * CT
                pltpu.async_copy(trans_v.at[d], out_hbm.at[pl.ds(dst, CT)], sem)
                return 0

            lax.fori_loop(0, D, dma_d, 0)

        def drain(trans_v, sem):
            pltpu.make_async_copy(
                out_hbm.at[pl.ds(0, D * CT)], trans_v, sem).wait()

        def pair(p, _):
            @pl.when(p > 0)
            def _():
                drain(trans_a, sem_a)

            compute_chunk(2 * p, trans_a)
            fire_chunk(2 * p, trans_a, sem_a)

            @pl.when(p > 0)
            def _():
                drain(trans_b, sem_b)

            compute_chunk(2 * p + 1, trans_b)
            fire_chunk(2 * p + 1, trans_b, sem_b)
            return 0

        lax.fori_loop(0, ST // 2, pair, 0)
        drain(trans_a, sem_a)
        compute_chunk(ST - 1, trans_a)
        fire_chunk(ST - 1, trans_a, sem_a)
        drain(trans_a, sem_a)
        drain(trans_b, sem_b)

    return sc_lookup


def kernel(indices, table):
    R, S = indices.shape
    V = VOCAB_SIZE
    D = table.shape[1]
    info = plsc.get_sparse_core_info()
    NC, NS, L = info.num_cores, info.num_subcores, info.num_lanes
    NW = NC * NS
    ST = S // 8
    sc_lookup = _build_sc_lookup(R, S, V, D, NC, NS, L)
    idx_r = (indices.reshape(NW, 128, ST, 8)
             .transpose(0, 2, 3, 1)
             .reshape(R * S))
    TBW = ((V + 1) * D + 7) // 8 * 8
    table_f = jnp.pad(table.reshape(-1), (0, TBW - (V + 1) * D))
    out5 = sc_lookup(idx_r, table_f).reshape(D, ST, NW, 8, 128)
    return out5.transpose(2, 4, 1, 3, 0).reshape(R, S, D)


# speedup vs baseline: 10.9002x; 1.1385x over previous
import functools

import jax
import jax.numpy as jnp
from jax import lax
from jax.experimental import pallas as pl
from jax.experimental.pallas import tpu as pltpu
from jax.experimental.pallas import tpu_sc as plsc

VOCAB_SIZE = 1000


def _build_sc_lookup(R, S, V, D, NC, NS, L):
    NW = NC * NS
    ST = S // 8
    CT = 1024
    TBW = ((V + 1) * D + 7) // 8 * 8

    mesh = plsc.VectorSubcoreMesh(core_axis_name="c", subcore_axis_name="s")

    @functools.partial(
        pl.kernel,
        mesh=mesh,
        compiler_params=pltpu.CompilerParams(
            use_tc_tiling_on_sc=False, needs_layout_passes=False),
        out_type=jax.ShapeDtypeStruct((D, ST * NW * CT), jnp.float32),
        scratch_types=[
            pltpu.VMEM((TBW,), jnp.float32),
            pltpu.VMEM((ST * CT,), jnp.int32),
            pltpu.VMEM((D, CT), jnp.float32),
            pltpu.VMEM((D, CT), jnp.float32),
            pltpu.SemaphoreType.DMA,
            pltpu.SemaphoreType.DMA,
        ],
    )
    def sc_lookup(idx_hbm, table_hbm, out_hbm, table_v, idx_v,
                  trans_a, trans_b, sem_a, sem_b):
        wid = lax.axis_index("s") * NC + lax.axis_index("c")
        pltpu.sync_copy(table_hbm, table_v)
        pltpu.sync_copy(idx_hbm.at[pl.ds(wid * ST * CT, ST * CT)], idx_v)

        def compute_chunk(st, trans_v):
            def kbody(k, _):
                v = idx_v[pl.ds(st * CT + k * L, L)]
                ok = (v >= 0) & (v < V)
                base = jnp.where(ok, v + 1, jnp.zeros_like(v)) * D
                vals = [plsc.load_gather(table_v, [base + d])
                        for d in range(D)]
                for d in range(D):
                    trans_v[d, pl.ds(k * L, L)] = vals[d]
                return 0

            lax.fori_loop(0, CT // L, kbody, 0)

        def fire_chunk(st, trans_v, sem):
            blk = (st * NW + wid) * CT
            pltpu.async_copy(trans_v, out_hbm.at[:, pl.ds(blk, CT)], sem)

        def drain(trans_v, sem):
            pltpu.make_async_copy(
                out_hbm.at[:, pl.ds(0, CT)], trans_v, sem).wait()

        def pair(p, _):
            @pl.when(p > 0)
            def _():
                drain(trans_a, sem_a)

            compute_chunk(2 * p, trans_a)
            fire_chunk(2 * p, trans_a, sem_a)

            @pl.when(p > 0)
            def _():
                drain(trans_b, sem_b)

            compute_chunk(2 * p + 1, trans_b)
            fire_chunk(2 * p + 1, trans_b, sem_b)
            return 0

        lax.fori_loop(0, ST // 2, pair, 0)
        drain(trans_a, sem_a)
        compute_chunk(ST - 1, trans_a)
        fire_chunk(ST - 1, trans_a, sem_a)
        drain(trans_a, sem_a)
        drain(trans_b, sem_b)

    return sc_lookup


def kernel(indices, table):
    R, S = indices.shape
    V = VOCAB_SIZE
    D = table.shape[1]
    info = plsc.get_sparse_core_info()
    NC, NS, L = info.num_cores, info.num_subcores, info.num_lanes
    NW = NC * NS
    ST = S // 8
    sc_lookup = _build_sc_lookup(R, S, V, D, NC, NS, L)
    idx_r = (indices.reshape(NW, 128, ST, 8)
             .transpose(0, 2, 3, 1)
             .reshape(R * S))
    TBW = ((V + 1) * D + 7) // 8 * 8
    table_f = jnp.pad(table.reshape(-1), (0, TBW - (V + 1) * D))
    out5 = sc_lookup(idx_r, table_f).reshape(D, ST, NW, 8, 128)
    return out5.transpose(2, 4, 1, 3, 0).reshape(R, S, D)


# speedup vs baseline: 11.0014x; 1.0093x over previous
import functools

import jax
import jax.numpy as jnp
from jax import lax
from jax.experimental import pallas as pl
from jax.experimental.pallas import tpu as pltpu
from jax.experimental.pallas import tpu_sc as plsc

VOCAB_SIZE = 1000


def _build_sc_lookup(R, S, V, D, NC, NS, L):
    NW = NC * NS
    ST = S // 8
    CT = 1024
    TBW = ((V + 1) * D + 7) // 8 * 8

    mesh = plsc.VectorSubcoreMesh(core_axis_name="c", subcore_axis_name="s")

    @functools.partial(
        pl.kernel,
        mesh=mesh,
        compiler_params=pltpu.CompilerParams(
            use_tc_tiling_on_sc=False, needs_layout_passes=False),
        out_type=jax.ShapeDtypeStruct((D, ST * NW * CT), jnp.float32),
        scratch_types=[
            pltpu.VMEM((TBW,), jnp.float32),
            pltpu.VMEM((ST * CT,), jnp.int32),
            pltpu.VMEM((D, CT), jnp.float32),
            pltpu.VMEM((D, CT), jnp.float32),
            pltpu.SemaphoreType.DMA,
            pltpu.SemaphoreType.DMA,
        ],
    )
    def sc_lookup(idx_hbm, table_hbm, out_hbm, table_v, idx_v,
                  trans_a, trans_b, sem_a, sem_b):
        wid = lax.axis_index("s") * NC + lax.axis_index("c")
        pltpu.sync_copy(table_hbm, table_v)
        pltpu.sync_copy(idx_hbm.at[pl.ds(wid * ST * CT, ST * CT)], idx_v)

        def compute_chunk(st, trans_v):
            def kbody(k, _):
                v = idx_v[pl.ds(st * CT + k * L, L)]
                ok = (v >= 0) & (v < V)
                base = jnp.where(ok, v + 1, jnp.zeros_like(v)) * D
                vals = [plsc.load_gather(table_v, [base + d])
                        for d in range(D)]
                for d in range(D):
                    trans_v[d, pl.ds(k * L, L)] = vals[d]
                return 0

            lax.fori_loop(0, CT // L, kbody, 0, unroll=2)

        def fire_chunk(st, trans_v, sem):
            blk = (st * NW + wid) * CT
            pltpu.async_copy(trans_v, out_hbm.at[:, pl.ds(blk, CT)], sem)

        def drain(trans_v, sem):
            pltpu.make_async_copy(
                out_hbm.at[:, pl.ds(0, CT)], trans_v, sem).wait()

        def pair(p, _):
            @pl.when(p > 0)
            def _():
                drain(trans_a, sem_a)

            compute_chunk(2 * p, trans_a)
            fire_chunk(2 * p, trans_a, sem_a)

            @pl.when(p > 0)
            def _():
                drain(trans_b, sem_b)

            compute_chunk(2 * p + 1, trans_b)
            fire_chunk(2 * p + 1, trans_b, sem_b)
            return 0

        lax.fori_loop(0, ST // 2, pair, 0)
        drain(trans_a, sem_a)
        compute_chunk(ST - 1, trans_a)
        fire_chunk(ST - 1, trans_a, sem_a)
        drain(trans_a, sem_a)
        drain(trans_b, sem_b)

    return sc_lookup


def kernel(indices, table):
    R, S = indices.shape
    V = VOCAB_SIZE
    D = table.shape[1]
    info = plsc.get_sparse_core_info()
    NC, NS, L = info.num_cores, info.num_subcores, info.num_lanes
    NW = NC * NS
    ST = S // 8
    sc_lookup = _build_sc_lookup(R, S, V, D, NC, NS, L)
    idx_r = (indices.reshape(NW, 128, ST, 8)
             .transpose(0, 2, 3, 1)
             .reshape(R * S))
    TBW = ((V + 1) * D + 7) // 8 * 8
    table_f = jnp.pad(table.reshape(-1), (0, TBW - (V + 1) * D))
    out5 = sc_lookup(idx_r, table_f).reshape(D, ST, NW, 8, 128)
    return out5.transpose(2, 4, 1, 3, 0).reshape(R, S, D)


# speedup vs baseline: 11.6498x; 1.0589x over previous
import functools

import jax
import jax.numpy as jnp
from jax import lax
from jax.experimental import pallas as pl
from jax.experimental.pallas import tpu as pltpu
from jax.experimental.pallas import tpu_sc as plsc

VOCAB_SIZE = 1000


def _build_sc_lookup(R, S, V, D, NC, NS, L):
    NW = NC * NS
    ST = S // 8
    CT = 1024
    TBW = ((V + 1) * D + 7) // 8 * 8

    mesh = plsc.VectorSubcoreMesh(core_axis_name="c", subcore_axis_name="s")

    @functools.partial(
        pl.kernel,
        mesh=mesh,
        compiler_params=pltpu.CompilerParams(
            use_tc_tiling_on_sc=False, needs_layout_passes=False),
        out_type=jax.ShapeDtypeStruct((D, ST * NW * CT), jnp.float32),
        scratch_types=[
            pltpu.VMEM((TBW,), jnp.float32),
            pltpu.VMEM((ST, CT), jnp.int32),
            pltpu.VMEM((D, CT), jnp.float32),
            pltpu.VMEM((D, CT), jnp.float32),
            pltpu.SemaphoreType.DMA,
            pltpu.SemaphoreType.DMA,
        ],
    )
    def sc_lookup(idx_hbm, table_hbm, out_hbm, table_v, idx_v,
                  trans_a, trans_b, sem_a, sem_b):
        wid = lax.axis_index("s") * NC + lax.axis_index("c")
        pltpu.sync_copy(table_hbm, table_v)
        pltpu.sync_copy(idx_hbm.at[:, pl.ds(wid * CT, CT)], idx_v)

        def compute_chunk(st, trans_v):
            def kbody(k, _):
                v = idx_v[st, pl.ds(k * L, L)]
                ok = (v >= 0) & (v < V)
                base = jnp.where(ok, v + 1, jnp.zeros_like(v)) * D
                vals = [plsc.load_gather(table_v, [base + d])
                        for d in range(D)]
                for d in range(D):
                    trans_v[d, pl.ds(k * L, L)] = vals[d]
                return 0

            lax.fori_loop(0, CT // L, kbody, 0, unroll=2)

        def fire_chunk(st, trans_v, sem):
            blk = (st * NW + wid) * CT
            pltpu.async_copy(trans_v, out_hbm.at[:, pl.ds(blk, CT)], sem)

        def drain(trans_v, sem):
            pltpu.make_async_copy(
                out_hbm.at[:, pl.ds(0, CT)], trans_v, sem).wait()

        def pair(p, _):
            @pl.when(p > 0)
            def _():
                drain(trans_a, sem_a)

            compute_chunk(2 * p, trans_a)
            fire_chunk(2 * p, trans_a, sem_a)

            @pl.when(p > 0)
            def _():
                drain(trans_b, sem_b)

            compute_chunk(2 * p + 1, trans_b)
            fire_chunk(2 * p + 1, trans_b, sem_b)
            return 0

        lax.fori_loop(0, ST // 2, pair, 0)
        drain(trans_a, sem_a)
        compute_chunk(ST - 1, trans_a)
        fire_chunk(ST - 1, trans_a, sem_a)
        drain(trans_a, sem_a)
        drain(trans_b, sem_b)

    return sc_lookup


def kernel(indices, table):
    R, S = indices.shape
    V = VOCAB_SIZE
    D = table.shape[1]
    info = plsc.get_sparse_core_info()
    NC, NS, L = info.num_cores, info.num_subcores, info.num_lanes
    NW = NC * NS
    ST = S // 8
    sc_lookup = _build_sc_lookup(R, S, V, D, NC, NS, L)
    idx_r = (indices.reshape(NW, 128, ST, 8)
             .transpose(2, 0, 3, 1)
             .reshape(ST, NW * 128 * 8))
    TBW = ((V + 1) * D + 7) // 8 * 8
    table_f = jnp.pad(table.reshape(-1), (0, TBW - (V + 1) * D))
    out5 = sc_lookup(idx_r, table_f).reshape(D, ST, NW, 8, 128)
    return out5.transpose(2, 4, 1, 3, 0).reshape(R, S, D)
